# baseline (speedup 1.0000x reference)
import functools

import jax
import jax.numpy as jnp
from jax import lax
from jax.experimental import pallas as pl
from jax.experimental.pallas import tpu as pltpu
from jax.experimental.pallas import tpu_sc as plsc

HIDDEN = 768
LANES = 16
KCHUNKS = HIDDEN // LANES
EPS = 1e-12


def _rsqrt_vec(x):
    xi = lax.bitcast_convert_type(x, jnp.int32)
    yi = jnp.int32(0x5F3759DF) - lax.shift_right_arithmetic(xi, 1)
    y = lax.bitcast_convert_type(yi, jnp.float32)
    for _ in range(3):
        y = y * (1.5 - 0.5 * x * y * y)
    return y


def _make_sc_kernel(n_tokens, seq_len, tpw, T):
    mesh = plsc.VectorSubcoreMesh(core_axis_name="c", subcore_axis_name="s")
    nc = 2
    nchunks = tpw // T

    @functools.partial(
        pl.kernel,
        mesh=mesh,
        compiler_params=pltpu.CompilerParams(needs_layout_passes=False),
        out_type=jax.ShapeDtypeStruct((n_tokens, HIDDEN), jnp.float32),
        scratch_types=[
            pltpu.VMEM((tpw,), jnp.int32),
            pltpu.VMEM((T,), jnp.int32),
            pltpu.VMEM((T,), jnp.int32),
            pltpu.VMEM((T, HIDDEN), jnp.float32),
            pltpu.VMEM((T, HIDDEN), jnp.float32),
            pltpu.VMEM((HIDDEN,), jnp.float32),
            pltpu.VMEM((HIDDEN,), jnp.float32),
            pltpu.SemaphoreType.DMA,
            pltpu.SemaphoreType.DMA,
        ],
    )
    def k(ids_hbm, word_hbm, pos_hbm, gamma_hbm, beta_hbm, out_hbm,
          idx_v, widx, pidx, wbuf, pbuf, gv, bv, sem_w, sem_p):
        wid = lax.axis_index("s") * nc + lax.axis_index("c")
        base_tok = wid * tpw
        base_s = lax.rem(base_tok, seq_len)

        pltpu.sync_copy(ids_hbm.at[pl.ds(base_tok, tpw)], idx_v)
        pltpu.sync_copy(gamma_hbm, gv)
        pltpu.sync_copy(beta_hbm, bv)

        lanes = lax.iota(jnp.int32, LANES)

        def chunk_body(c, carry):
            coff = c * T
            for g in range(T // LANES):
                ids = idx_v[pl.ds(coff + g * LANES, LANES)]
                widx[pl.ds(g * LANES, LANES)] = ids
                s_pos = base_s + coff + g * LANES + lanes + 1
                pidx[pl.ds(g * LANES, LANES)] = jnp.where(
                    ids == 0, jnp.zeros((LANES,), jnp.int32), s_pos)
            cp_w = pltpu.async_copy(word_hbm.at[widx], wbuf, sem_w)
            cp_p = pltpu.async_copy(pos_hbm.at[pidx], pbuf, sem_p)
            cp_w.wait()
            cp_p.wait()

            def token_body(j, carry2):
                acc = jnp.zeros((LANES,), jnp.float32)
                acc2 = jnp.zeros((LANES,), jnp.float32)
                for kk in range(KCHUNKS):
                    sl = pl.ds(kk * LANES, LANES)
                    e = wbuf[j, sl] + pbuf[j, sl]
                    wbuf[j, sl] = e
                    acc = acc + e
                    acc2 = acc2 + e * e
                s1 = lax.broadcast(jnp.sum(acc), (LANES,))
                s2 = lax.broadcast(jnp.sum(acc2), (LANES,))
                meanv = s1 * (1.0 / HIDDEN)
                varv = jnp.maximum(s2 * (1.0 / HIDDEN) - meanv * meanv, 0.0)
                invv = _rsqrt_vec(varv + EPS)
                for kk in range(KCHUNKS):
                    sl = pl.ds(kk * LANES, LANES)
                    e = wbuf[j, sl]
                    wbuf[j, sl] = (e - meanv) * invv * gv[sl] + bv[sl]
                return carry2

            lax.fori_loop(0, T, token_body, 0)
            pltpu.sync_copy(wbuf, out_hbm.at[pl.ds(base_tok + coff, T)])
            return carry

        lax.fori_loop(0, nchunks, chunk_body, 0)

    return k


def kernel(input_ids, word_emb, pos_table, gamma, beta):
    B, S = input_ids.shape
    n_tokens = B * S
    n_workers = 32
    tpw = n_tokens // n_workers
    k = _make_sc_kernel(n_tokens, S, tpw, T=32)
    ids_flat = input_ids.reshape(-1)
    out = k(ids_flat, word_emb, pos_table, gamma, beta)
    return out.reshape(B, S, HIDDEN)


# speedup vs baseline: 1.1455x; 1.1455x over previous
import functools

import jax
import jax.numpy as jnp
from jax import lax
from jax.experimental import pallas as pl
from jax.experimental.pallas import tpu as pltpu
from jax.experimental.pallas import tpu_sc as plsc

HIDDEN = 768
LANES = 16
KCHUNKS = HIDDEN // LANES
EPS = 1e-12


def _rsqrt_vec(x):
    xi = lax.bitcast_convert_type(x, jnp.int32)
    yi = jnp.int32(0x5F3759DF) - lax.shift_right_arithmetic(xi, 1)
    y = lax.bitcast_convert_type(yi, jnp.float32)
    for _ in range(3):
        y = y * (1.5 - 0.5 * x * y * y)
    return y


def _make_sc_kernel(n_tokens, seq_len, tpw, T):
    mesh = plsc.VectorSubcoreMesh(core_axis_name="c", subcore_axis_name="s")
    nc = 2
    nchunks = tpw // T

    @functools.partial(
        pl.kernel,
        mesh=mesh,
        compiler_params=pltpu.CompilerParams(needs_layout_passes=False),
        out_type=jax.ShapeDtypeStruct((n_tokens, HIDDEN), jnp.float32),
        scratch_types=[
            pltpu.VMEM((tpw,), jnp.int32),
            pltpu.VMEM((tpw,), jnp.int32),
            pltpu.VMEM((2, T, HIDDEN), jnp.float32),
            pltpu.VMEM((2, T, HIDDEN), jnp.float32),
            pltpu.VMEM((2, T, HIDDEN), jnp.float32),
            pltpu.VMEM((HIDDEN,), jnp.float32),
            pltpu.VMEM((HIDDEN,), jnp.float32),
            pltpu.SemaphoreType.DMA,
            pltpu.SemaphoreType.DMA,
            pltpu.SemaphoreType.DMA,
            pltpu.SemaphoreType.DMA,
            pltpu.SemaphoreType.DMA,
            pltpu.SemaphoreType.DMA,
        ],
    )
    def k(ids_hbm, word_hbm, pos_hbm, gamma_hbm, beta_hbm, out_hbm,
          idx_v, pidx, wbuf, pbuf, obuf, gv, bv,
          sem_w0, sem_w1, sem_p0, sem_p1, sem_o0, sem_o1):
        sem_w = (sem_w0, sem_w1)
        sem_p = (sem_p0, sem_p1)
        sem_o = (sem_o0, sem_o1)
        wid = lax.axis_index("s") * nc + lax.axis_index("c")
        base_tok = wid * tpw
        base_s = lax.rem(base_tok, seq_len)

        pltpu.sync_copy(ids_hbm.at[pl.ds(base_tok, tpw)], idx_v)
        pltpu.sync_copy(gamma_hbm, gv)
        pltpu.sync_copy(beta_hbm, bv)

        lanes = lax.iota(jnp.int32, LANES)
        for g in range(tpw // LANES):
            ids = idx_v[pl.ds(g * LANES, LANES)]
            s_pos = base_s + g * LANES + lanes + 1
            pidx[pl.ds(g * LANES, LANES)] = jnp.where(
                ids == 0, jnp.zeros((LANES,), jnp.int32), s_pos)

        def issue_gathers(c, b):
            coff = c * T
            cw = pltpu.async_copy(
                word_hbm.at[idx_v.at[pl.ds(coff, T)]], wbuf.at[b], sem_w[b])
            cp = pltpu.async_copy(
                pos_hbm.at[pidx.at[pl.ds(coff, T)]], pbuf.at[b], sem_p[b])
            return cw, cp

        def wait_gathers(c, b):
            coff = c * T
            pltpu.make_async_copy(
                word_hbm.at[idx_v.at[pl.ds(coff, T)]], wbuf.at[b],
                sem_w[b]).wait()
            pltpu.make_async_copy(
                pos_hbm.at[pidx.at[pl.ds(coff, T)]], pbuf.at[b],
                sem_p[b]).wait()

        def out_slice(c):
            return out_hbm.at[pl.ds(base_tok + c * T, T)]

        def compute_chunk(b):
            def token_body(j, carry2):
                acc = jnp.zeros((LANES,), jnp.float32)
                acc2 = jnp.zeros((LANES,), jnp.float32)
                for kk in range(KCHUNKS):
                    sl = pl.ds(kk * LANES, LANES)
                    e = wbuf[b, j, sl] + pbuf[b, j, sl]
                    wbuf[b, j, sl] = e
                    acc = acc + e
                    acc2 = acc2 + e * e
                s1 = lax.broadcast(jnp.sum(acc), (LANES,))
                s2 = lax.broadcast(jnp.sum(acc2), (LANES,))
                meanv = s1 * (1.0 / HIDDEN)
                varv = jnp.maximum(s2 * (1.0 / HIDDEN) - meanv * meanv, 0.0)
                invv = _rsqrt_vec(varv + EPS)
                for kk in range(KCHUNKS):
                    sl = pl.ds(kk * LANES, LANES)
                    e = wbuf[b, j, sl]
                    obuf[b, j, sl] = (e - meanv) * invv * gv[sl] + bv[sl]
                return carry2

            lax.fori_loop(0, T, token_body, 0)

        issue_gathers(0, 0)
        issue_gathers(1, 1)

        def outer_body(ci, carry):
            for b in (0, 1):
                c = 2 * ci + b
                wait_gathers(c, b)

                @pl.when(c >= 2)
                def _():
                    pltpu.make_async_copy(
                        obuf.at[b], out_slice(c - 2), sem_o[b]).wait()

                compute_chunk(b)
                pltpu.async_copy(obuf.at[b], out_slice(c), sem_o[b])

                @pl.when(c + 2 < nchunks)
                def _():
                    issue_gathers(c + 2, b)
            return carry

        lax.fori_loop(0, nchunks // 2, outer_body, 0)
        pltpu.make_async_copy(obuf.at[0], out_slice(nchunks - 2), sem_o0).wait()
        pltpu.make_async_copy(obuf.at[1], out_slice(nchunks - 1), sem_o1).wait()

    return k


def kernel(input_ids, word_emb, pos_table, gamma, beta):
    B, S = input_ids.shape
    n_tokens = B * S
    n_workers = 32
    tpw = n_tokens // n_workers
    k = _make_sc_kernel(n_tokens, S, tpw, T=16)
    ids_flat = input_ids.reshape(-1)
    out = k(ids_flat, word_emb, pos_table, gamma, beta)
    return out.reshape(B, S, HIDDEN)
